# baseline (speedup 1.0000x reference)
import functools

import jax
import jax.numpy as jnp
from jax.experimental import pallas as pl
from jax.experimental.pallas import tpu as pltpu


def _fused_kernel(labels_blk_ref, feat_ref, tab_ref, scores_ref, pos_ref,
                  pids_ref, npos_ref, off_scr, hi_scr, lo_scr, *, n_steps):
    i = pl.program_id(0)
    tn, f = feat_ref.shape
    nr = pids_ref.shape[0]
    n = nr * 128
    k = scores_ref.shape[1]
    k_pad = tab_ref.shape[0]
    tab = tab_ref[...]

    @pl.when(i == 0)
    def _init():
        off_scr[...] = jnp.zeros((1, 1), jnp.int32)
        hi_scr[...] = jnp.zeros((nr, 128), jnp.float32)
        lo_scr[...] = jnp.zeros((nr, 128), jnp.float32)
        pos_ref[...] = jnp.zeros((n, f), jnp.float32)

    feat = feat_ref[...].astype(jnp.bfloat16)
    s_full = jax.lax.dot_general(
        feat, tab, (((1,), (1,)), ((), ())),
        preferred_element_type=jnp.float32)
    scores_ref[...] = s_full[:, :k]

    labels_blk = labels_blk_ref[...]
    m = (labels_blk > 0).astype(jnp.float32)
    incl = m
    shift = 1
    while shift < tn:
        top = jnp.zeros((shift, 1), jnp.float32)
        incl = incl + jnp.concatenate([top, incl[:tn - shift]], axis=0)
        shift *= 2
    off = off_scr[...]
    slot_blk = jnp.where(labels_blk > 0,
                         off + incl.astype(jnp.int32) - 1, -1)
    off_scr[...] = off + incl[tn - 1:, :].astype(jnp.int32)

    col = jax.lax.broadcasted_iota(jnp.int32, (tn, k_pad), 1)
    g = jnp.dot((labels_blk == col).astype(jnp.bfloat16), tab,
                preferred_element_type=jnp.float32)

    s_iota = jax.lax.broadcasted_iota(jnp.int32, (tn, n), 1)
    m_t = (slot_blk == s_iota).astype(jnp.bfloat16)
    pos_ref[...] += jax.lax.dot_general(
        m_t, g.astype(jnp.bfloat16), (((0,), (0,)), ((), ())),
        preferred_element_type=jnp.float32)

    sr = jnp.right_shift(slot_blk, 7)
    sc = jnp.bitwise_and(slot_blk, 127)
    r_iota = jax.lax.broadcasted_iota(jnp.int32, (tn, nr), 1)
    c_iota = jax.lax.broadcasted_iota(jnp.int32, (tn, 128), 1)
    a_sr = (sr == r_iota).astype(jnp.bfloat16)
    b_c = (sc == c_iota).astype(jnp.bfloat16)
    hi = jnp.right_shift(labels_blk, 8).astype(jnp.bfloat16)
    lo = jnp.bitwise_and(labels_blk, 255).astype(jnp.bfloat16)
    hi_scr[...] += jax.lax.dot_general(
        a_sr, b_c * hi, (((0,), (0,)), ((), ())),
        preferred_element_type=jnp.float32)
    lo_scr[...] += jax.lax.dot_general(
        a_sr, b_c * lo, (((0,), (0,)), ((), ())),
        preferred_element_type=jnp.float32)

    @pl.when(i == n_steps - 1)
    def _emit():
        n_pos = off_scr[...]
        npos_ref[...] = n_pos
        pids_ref[...] = (hi_scr[...] * 256.0 + lo_scr[...]).astype(jnp.int32)
        out_row = jax.lax.broadcasted_iota(jnp.int32, (n, 1), 0)
        pad = out_row >= n_pos
        pos_ref[...] = jnp.where(pad, tab[0:1, :].astype(jnp.float32),
                                 pos_ref[...])


def _pick_tn(n):
    for tn in (256, 128, 64, 32, 16, 8):
        if n % tn == 0:
            return tn
    return n


@jax.jit
def _device_fn(features, pid_labels, lookup_table):
    N, F = features.shape
    K, F2 = lookup_table.shape
    assert F == F2
    assert N % 128 == 0

    TN = _pick_tn(N)
    n_steps = N // TN
    NR = N // 128
    K_pad = ((K + 127) // 128) * 128

    tab = jnp.pad(lookup_table.astype(jnp.bfloat16), ((0, K_pad - K), (0, 0)))
    labels_col = pid_labels.astype(jnp.int32).reshape(N, 1)

    scores, pos_feats_pad, pids, npos = pl.pallas_call(
        functools.partial(_fused_kernel, n_steps=n_steps),
        out_shape=(
            jax.ShapeDtypeStruct((N, K), jnp.float32),
            jax.ShapeDtypeStruct((N, F), jnp.float32),
            jax.ShapeDtypeStruct((NR, 128), jnp.int32),
            jax.ShapeDtypeStruct((1, 1), jnp.int32),
        ),
        grid=(n_steps,),
        in_specs=[
            pl.BlockSpec((TN, 1), lambda i: (i, 0)),
            pl.BlockSpec((TN, F), lambda i: (i, 0)),
            pl.BlockSpec((K_pad, F), lambda i: (0, 0)),
        ],
        out_specs=(
            pl.BlockSpec((TN, K), lambda i: (i, 0)),
            pl.BlockSpec((N, F), lambda i: (0, 0)),
            pl.BlockSpec((NR, 128), lambda i: (0, 0)),
            pl.BlockSpec((1, 1), lambda i: (0, 0)),
        ),
        scratch_shapes=[
            pltpu.VMEM((1, 1), jnp.int32),
            pltpu.VMEM((NR, 128), jnp.float32),
            pltpu.VMEM((NR, 128), jnp.float32),
        ],
        compiler_params=pltpu.CompilerParams(
            dimension_semantics=("arbitrary",)),
    )(labels_col, features, tab)

    return scores, pos_feats_pad, pids.reshape(N), npos.reshape(())


def kernel(features, pid_labels, lookup_table):
    return _device_fn(features, pid_labels, lookup_table)


# speedup vs baseline: 1.2023x; 1.2023x over previous
import functools

import jax
import jax.numpy as jnp
from jax.experimental import pallas as pl
from jax.experimental.pallas import tpu as pltpu


def _fused_kernel(labels_blk_ref, feat_ref, tab_ref, scores_ref, pos_ref,
                  pids_ref, npos_ref, off_scr, hi_scr, lo_scr, *, n_steps):
    i = pl.program_id(0)
    tn, f = feat_ref.shape
    nr = pids_ref.shape[0]
    n = nr * 128
    k = scores_ref.shape[1]
    k_pad = tab_ref.shape[0]
    tab = tab_ref[...]

    @pl.when(i == 0)
    def _init():
        off_scr[...] = jnp.zeros((1, 1), jnp.int32)
        hi_scr[...] = jnp.zeros((nr, 128), jnp.float32)
        lo_scr[...] = jnp.zeros((nr, 128), jnp.float32)
        pos_ref[...] = jnp.zeros((n, f), jnp.float32)

    feat = feat_ref[...].astype(jnp.bfloat16)
    s_full = jax.lax.dot_general(
        feat, tab, (((1,), (1,)), ((), ())),
        preferred_element_type=jnp.float32)
    scores_ref[...] = s_full[:, :k]

    labels_blk = labels_blk_ref[...]
    m = (labels_blk > 0).astype(jnp.float32)
    incl = m
    shift = 1
    while shift < tn:
        top = jnp.zeros((shift, 1), jnp.float32)
        incl = incl + jnp.concatenate([top, incl[:tn - shift]], axis=0)
        shift *= 2
    off = off_scr[...]
    slot_blk = jnp.where(labels_blk > 0,
                         off + incl.astype(jnp.int32) - 1, -1)
    off_scr[...] = off + incl[tn - 1:, :].astype(jnp.int32)

    col = jax.lax.broadcasted_iota(jnp.int32, (tn, k_pad), 1)
    g = jnp.dot((labels_blk == col).astype(jnp.bfloat16), tab,
                preferred_element_type=jnp.float32)

    wn = min(tn + 128, n)
    start = jnp.minimum((off[0, 0] // 8) * 8, n - wn)
    s_iota = jax.lax.broadcasted_iota(jnp.int32, (tn, wn), 1)
    m_t = ((slot_blk - start) == s_iota).astype(jnp.bfloat16)
    pos_ref[pl.ds(start, wn), :] += jax.lax.dot_general(
        m_t, g.astype(jnp.bfloat16), (((0,), (0,)), ((), ())),
        preferred_element_type=jnp.float32)

    sr = jnp.right_shift(slot_blk, 7)
    sc = jnp.bitwise_and(slot_blk, 127)
    r_iota = jax.lax.broadcasted_iota(jnp.int32, (tn, nr), 1)
    c_iota = jax.lax.broadcasted_iota(jnp.int32, (tn, 128), 1)
    a_sr = (sr == r_iota).astype(jnp.bfloat16)
    b_c = (sc == c_iota).astype(jnp.bfloat16)
    hi = jnp.right_shift(labels_blk, 8).astype(jnp.bfloat16)
    lo = jnp.bitwise_and(labels_blk, 255).astype(jnp.bfloat16)
    hi_scr[...] += jax.lax.dot_general(
        a_sr, b_c * hi, (((0,), (0,)), ((), ())),
        preferred_element_type=jnp.float32)
    lo_scr[...] += jax.lax.dot_general(
        a_sr, b_c * lo, (((0,), (0,)), ((), ())),
        preferred_element_type=jnp.float32)

    @pl.when(i == n_steps - 1)
    def _emit():
        n_pos = off_scr[...]
        npos_ref[...] = n_pos
        pids_ref[...] = (hi_scr[...] * 256.0 + lo_scr[...]).astype(jnp.int32)
        out_row = jax.lax.broadcasted_iota(jnp.int32, (n, 1), 0)
        pad = out_row >= n_pos
        pos_ref[...] = jnp.where(pad, tab[0:1, :].astype(jnp.float32),
                                 pos_ref[...])


def _pick_tn(n):
    for tn in (256, 128, 64, 32, 16, 8):
        if n % tn == 0:
            return tn
    return n


@jax.jit
def _device_fn(features, pid_labels, lookup_table):
    N, F = features.shape
    K, F2 = lookup_table.shape
    assert F == F2
    assert N % 128 == 0

    TN = _pick_tn(N)
    n_steps = N // TN
    NR = N // 128
    K_pad = ((K + 127) // 128) * 128

    tab = jnp.pad(lookup_table.astype(jnp.bfloat16), ((0, K_pad - K), (0, 0)))
    labels_col = pid_labels.astype(jnp.int32).reshape(N, 1)

    scores, pos_feats_pad, pids, npos = pl.pallas_call(
        functools.partial(_fused_kernel, n_steps=n_steps),
        out_shape=(
            jax.ShapeDtypeStruct((N, K), jnp.float32),
            jax.ShapeDtypeStruct((N, F), jnp.float32),
            jax.ShapeDtypeStruct((NR, 128), jnp.int32),
            jax.ShapeDtypeStruct((1, 1), jnp.int32),
        ),
        grid=(n_steps,),
        in_specs=[
            pl.BlockSpec((TN, 1), lambda i: (i, 0)),
            pl.BlockSpec((TN, F), lambda i: (i, 0)),
            pl.BlockSpec((K_pad, F), lambda i: (0, 0)),
        ],
        out_specs=(
            pl.BlockSpec((TN, K), lambda i: (i, 0)),
            pl.BlockSpec((N, F), lambda i: (0, 0)),
            pl.BlockSpec((NR, 128), lambda i: (0, 0)),
            pl.BlockSpec((1, 1), lambda i: (0, 0)),
        ),
        scratch_shapes=[
            pltpu.VMEM((1, 1), jnp.int32),
            pltpu.VMEM((NR, 128), jnp.float32),
            pltpu.VMEM((NR, 128), jnp.float32),
        ],
        compiler_params=pltpu.CompilerParams(
            dimension_semantics=("arbitrary",)),
    )(labels_col, features, tab)

    return scores, pos_feats_pad, pids.reshape(N), npos.reshape(())


def kernel(features, pid_labels, lookup_table):
    return _device_fn(features, pid_labels, lookup_table)


# speedup vs baseline: 1.2338x; 1.0262x over previous
import jax
import jax.numpy as jnp
from jax.experimental import pallas as pl
from jax.experimental.pallas import tpu as pltpu


def _fused_kernel(idx_ref, feat_ref, tab_ref, scores_ref, pos_ref):
    feat = feat_ref[...].astype(jnp.bfloat16)
    k = scores_ref.shape[1]
    row = jax.lax.broadcasted_iota(jnp.int32, tab_ref.shape, 0)
    tab = jnp.where(row < k, tab_ref[...], 0.0).astype(jnp.bfloat16)

    s_full = jax.lax.dot_general(
        feat, tab, (((1,), (1,)), ((), ())),
        preferred_element_type=jnp.float32)
    scores_ref[...] = s_full[:, :k]

    col = jax.lax.broadcasted_iota(jnp.int32, (feat.shape[0], tab.shape[0]), 1)
    onehot = (idx_ref[...] == col).astype(jnp.bfloat16)
    pos_ref[...] = jnp.dot(onehot, tab, preferred_element_type=jnp.float32)


def _pick_tn(n):
    for tn in (256, 128, 64, 32, 16, 8):
        if n % tn == 0:
            return tn
    return n


@jax.jit
def _device_fn(features, pid_labels, lookup_table):
    N, F = features.shape
    K, F2 = lookup_table.shape
    assert F == F2

    labels = pid_labels.astype(jnp.int32)
    mask = labels > 0
    n_pos = jnp.sum(mask.astype(jnp.int32))
    _, sorted_labels = jax.lax.sort_key_val(
        (~mask).astype(jnp.int32), labels, is_stable=True)
    pos_pids_pad = jnp.where(jnp.arange(N) < n_pos, sorted_labels, 0)

    TN = _pick_tn(N)
    K_pad = ((K + 127) // 128) * 128

    idx_col = pos_pids_pad.reshape(N, 1)

    scores, pos_feats_pad = pl.pallas_call(
        _fused_kernel,
        out_shape=(
            jax.ShapeDtypeStruct((N, K), jnp.float32),
            jax.ShapeDtypeStruct((N, F), jnp.float32),
        ),
        grid=(N // TN,),
        in_specs=[
            pl.BlockSpec((TN, 1), lambda i: (i, 0)),
            pl.BlockSpec((TN, F), lambda i: (i, 0)),
            pl.BlockSpec((K_pad, F), lambda i: (0, 0)),
        ],
        out_specs=(
            pl.BlockSpec((TN, K), lambda i: (i, 0)),
            pl.BlockSpec((TN, F), lambda i: (i, 0)),
        ),
        compiler_params=pltpu.CompilerParams(
            dimension_semantics=("parallel",)),
    )(idx_col, features, lookup_table)

    return scores, pos_feats_pad, pos_pids_pad, n_pos


def kernel(features, pid_labels, lookup_table):
    return _device_fn(features, pid_labels, lookup_table)
